# baseline (device time: 25573 ns/iter reference)
import jax
import jax.numpy as jnp
from jax import lax
from jax.experimental import pallas as pl
from jax.experimental.pallas import tpu as pltpu

N_DEV = 16
M = 512
N = 512
ROWS = M // N_DEV


def kernel(x, W1, W2):
    def body(
        x_ref,
        w1_ref,
        w2_ref,
        out_ref,
        staged_ref,
        p1buf_ref,
        ag_staged_ref,
        ag_buf_ref,
        ss1, rs1, ss2, rs2,
    ):
        my = lax.axis_index("i")

        barrier_sem = pltpu.get_barrier_semaphore()
        for c in range(N_DEV):
            @pl.when(c != my)
            def _():
                pl.semaphore_signal(
                    barrier_sem,
                    inc=1,
                    device_id=(c,),
                    device_id_type=pl.DeviceIdType.MESH,
                )

        def p1_desc(c):
            return pltpu.make_async_remote_copy(
                src_ref=staged_ref.at[c],
                dst_ref=p1buf_ref.at[my],
                send_sem=ss1.at[c],
                recv_sem=rs1.at[my],
                device_id=(c,),
                device_id_type=pl.DeviceIdType.MESH,
            )

        w1 = w1_ref[...].astype(jnp.bfloat16)
        w2 = w2_ref[...].astype(jnp.bfloat16)
        n_groups = 4
        g_rows = M // n_groups
        g_chunks = N_DEV // n_groups
        for g in range(n_groups):
            xb = x_ref[pl.ds(g * g_rows, g_rows), :].astype(jnp.bfloat16)
            h = jnp.dot(xb, w1, preferred_element_type=jnp.float32)
            hb = jnp.maximum(h, 0.0).astype(jnp.bfloat16)
            part = jnp.dot(hb, w2, preferred_element_type=jnp.float32)
            staged_ref[pl.ds(g * g_chunks, g_chunks)] = (
                part.astype(jnp.bfloat16).reshape(g_chunks, ROWS, N)
            )
            if g == 0:
                pl.semaphore_wait(barrier_sem, N_DEV - 1)
            for c in range(g * g_chunks, (g + 1) * g_chunks):
                @pl.when(c != my)
                def _():
                    p1_desc(c).start()
        p1buf_ref[pl.ds(my, 1)] = staged_ref[pl.ds(my, 1)]

        def p1_recv_desc(s):
            return pltpu.make_async_remote_copy(
                src_ref=staged_ref.at[s],
                dst_ref=p1buf_ref.at[s],
                send_sem=ss1.at[s],
                recv_sem=rs1.at[s],
                device_id=(s,),
                device_id_type=pl.DeviceIdType.MESH,
            )

        for s in range(N_DEV):
            @pl.when(s != my)
            def _():
                p1_recv_desc(s).wait_recv()

        chunk = jnp.sum(p1buf_ref[...].astype(jnp.float32), axis=0)
        ag_staged_ref[...] = chunk.astype(jnp.bfloat16)

        def p2_desc(c):
            return pltpu.make_async_remote_copy(
                src_ref=ag_staged_ref,
                dst_ref=ag_buf_ref.at[my],
                send_sem=ss2.at[c],
                recv_sem=rs2.at[my],
                device_id=(c,),
                device_id_type=pl.DeviceIdType.MESH,
            )

        for c in range(N_DEV):
            @pl.when(c != my)
            def _():
                p2_desc(c).start()
        ag_buf_ref[pl.ds(my, 1)] = ag_staged_ref[...].reshape(1, ROWS, N)

        def p2_recv_desc(s):
            return pltpu.make_async_remote_copy(
                src_ref=ag_staged_ref,
                dst_ref=ag_buf_ref.at[s],
                send_sem=ss2.at[s],
                recv_sem=rs2.at[s],
                device_id=(s,),
                device_id_type=pl.DeviceIdType.MESH,
            )

        for s in range(N_DEV):
            @pl.when(s != my)
            def _():
                p2_recv_desc(s).wait_recv()
            out_ref[pl.ds(s * ROWS, ROWS), :] = ag_buf_ref[s].astype(jnp.float32)

        for s in range(N_DEV):
            @pl.when(s != my)
            def _():
                p1_recv_desc(s).wait_send()
                p2_recv_desc(s).wait_send()

    return pl.pallas_call(
        body,
        out_shape=jax.ShapeDtypeStruct((M, N), jnp.float32),
        in_specs=[
            pl.BlockSpec(memory_space=pltpu.VMEM),
            pl.BlockSpec(memory_space=pltpu.VMEM),
            pl.BlockSpec(memory_space=pltpu.VMEM),
        ],
        out_specs=pl.BlockSpec(memory_space=pltpu.VMEM),
        scratch_shapes=[
            pltpu.VMEM((N_DEV, ROWS, N), jnp.bfloat16),
            pltpu.VMEM((N_DEV, ROWS, N), jnp.bfloat16),
            pltpu.VMEM((ROWS, N), jnp.bfloat16),
            pltpu.VMEM((N_DEV, ROWS, N), jnp.bfloat16),
            pltpu.SemaphoreType.DMA((N_DEV,)),
            pltpu.SemaphoreType.DMA((N_DEV,)),
            pltpu.SemaphoreType.DMA((N_DEV,)),
            pltpu.SemaphoreType.DMA((N_DEV,)),
        ],
        compiler_params=pltpu.CompilerParams(collective_id=0),
    )(x, W1, W2)


# device time: 24759 ns/iter; 1.0329x vs baseline; 1.0329x over previous
import jax
import jax.numpy as jnp
from jax import lax
from jax.experimental import pallas as pl
from jax.experimental.pallas import tpu as pltpu

N_DEV = 16
M = 512
N = 512
ROWS = M // N_DEV


def kernel(x, W1, W2):
    def body(
        x_ref,
        w1_ref,
        w2_ref,
        out_ref,
        staged_ref,
        p1buf_ref,
        ag_staged_ref,
        ag_buf_ref,
        ss1, rs1, ss2, rs2,
        entry_sems,
    ):
        my = lax.axis_index("i")

        barrier_sem = pltpu.get_barrier_semaphore()
        pl.semaphore_signal(barrier_sem, inc=1)
        pl.semaphore_wait(barrier_sem, 1)

        for c in range(N_DEV):
            @pl.when(c != my)
            def _():
                pl.semaphore_signal(
                    entry_sems.at[my],
                    inc=1,
                    device_id=(c,),
                    device_id_type=pl.DeviceIdType.MESH,
                )

        def p1_desc(c):
            return pltpu.make_async_remote_copy(
                src_ref=staged_ref.at[c],
                dst_ref=p1buf_ref.at[my],
                send_sem=ss1.at[c],
                recv_sem=rs1.at[my],
                device_id=(c,),
                device_id_type=pl.DeviceIdType.MESH,
            )

        w1 = w1_ref[...].astype(jnp.bfloat16)
        w2 = w2_ref[...].astype(jnp.bfloat16)
        n_groups = 4
        g_rows = M // n_groups
        g_chunks = N_DEV // n_groups
        for g in range(n_groups):
            xb = x_ref[pl.ds(g * g_rows, g_rows), :].astype(jnp.bfloat16)
            h = jnp.dot(xb, w1, preferred_element_type=jnp.float32)
            hb = jnp.maximum(h, 0.0).astype(jnp.bfloat16)
            part = jnp.dot(hb, w2, preferred_element_type=jnp.float32)
            staged_ref[pl.ds(g * g_chunks, g_chunks)] = (
                part.astype(jnp.bfloat16).reshape(g_chunks, ROWS, N)
            )
            for c in range(g * g_chunks, (g + 1) * g_chunks):
                @pl.when(c != my)
                def _():
                    pl.semaphore_wait(entry_sems.at[c], 1)
                    p1_desc(c).start()
        p1buf_ref[pl.ds(my, 1)] = staged_ref[pl.ds(my, 1)]

        def p1_recv_desc(s):
            return pltpu.make_async_remote_copy(
                src_ref=staged_ref.at[s],
                dst_ref=p1buf_ref.at[s],
                send_sem=ss1.at[s],
                recv_sem=rs1.at[s],
                device_id=(s,),
                device_id_type=pl.DeviceIdType.MESH,
            )

        for s in range(N_DEV):
            @pl.when(s != my)
            def _():
                p1_recv_desc(s).wait_recv()

        chunk = jnp.sum(p1buf_ref[...].astype(jnp.float32), axis=0)
        ag_staged_ref[...] = chunk.astype(jnp.bfloat16)

        def p2_desc(c):
            return pltpu.make_async_remote_copy(
                src_ref=ag_staged_ref,
                dst_ref=ag_buf_ref.at[my],
                send_sem=ss2.at[c],
                recv_sem=rs2.at[my],
                device_id=(c,),
                device_id_type=pl.DeviceIdType.MESH,
            )

        for c in range(N_DEV):
            @pl.when(c != my)
            def _():
                p2_desc(c).start()
        ag_buf_ref[pl.ds(my, 1)] = ag_staged_ref[...].reshape(1, ROWS, N)

        def p2_recv_desc(s):
            return pltpu.make_async_remote_copy(
                src_ref=ag_staged_ref,
                dst_ref=ag_buf_ref.at[s],
                send_sem=ss2.at[s],
                recv_sem=rs2.at[s],
                device_id=(s,),
                device_id_type=pl.DeviceIdType.MESH,
            )

        for s in range(N_DEV):
            @pl.when(s != my)
            def _():
                p2_recv_desc(s).wait_recv()
            out_ref[pl.ds(s * ROWS, ROWS), :] = ag_buf_ref[s].astype(jnp.float32)

        for s in range(N_DEV):
            @pl.when(s != my)
            def _():
                p1_recv_desc(s).wait_send()
                p2_recv_desc(s).wait_send()

    return pl.pallas_call(
        body,
        out_shape=jax.ShapeDtypeStruct((M, N), jnp.float32),
        in_specs=[
            pl.BlockSpec(memory_space=pltpu.VMEM),
            pl.BlockSpec(memory_space=pltpu.VMEM),
            pl.BlockSpec(memory_space=pltpu.VMEM),
        ],
        out_specs=pl.BlockSpec(memory_space=pltpu.VMEM),
        scratch_shapes=[
            pltpu.VMEM((N_DEV, ROWS, N), jnp.bfloat16),
            pltpu.VMEM((N_DEV, ROWS, N), jnp.bfloat16),
            pltpu.VMEM((ROWS, N), jnp.bfloat16),
            pltpu.VMEM((N_DEV, ROWS, N), jnp.bfloat16),
            pltpu.SemaphoreType.DMA((N_DEV,)),
            pltpu.SemaphoreType.DMA((N_DEV,)),
            pltpu.SemaphoreType.DMA((N_DEV,)),
            pltpu.SemaphoreType.DMA((N_DEV,)),
            pltpu.SemaphoreType.REGULAR((N_DEV,)),
        ],
        compiler_params=pltpu.CompilerParams(collective_id=0),
    )(x, W1, W2)


# device time: 24741 ns/iter; 1.0336x vs baseline; 1.0007x over previous
import jax
import jax.numpy as jnp
from jax import lax
from jax.experimental import pallas as pl
from jax.experimental.pallas import tpu as pltpu

N_DEV = 16
M = 512
N = 512
ROWS = M // N_DEV


def kernel(x, W1, W2):
    def body(
        x_ref,
        w1_ref,
        w2_ref,
        out_ref,
        staged_ref,
        p1buf_ref,
        ag_staged_ref,
        ag_buf_ref,
        ss1, rs1, ss2, rs2,
        entry_sems,
    ):
        my = lax.axis_index("i")

        barrier_sem = pltpu.get_barrier_semaphore()
        pl.semaphore_signal(barrier_sem, inc=1)
        pl.semaphore_wait(barrier_sem, 1)

        for c in range(N_DEV):
            @pl.when(c != my)
            def _():
                pl.semaphore_signal(
                    entry_sems.at[my],
                    inc=1,
                    device_id=(c,),
                    device_id_type=pl.DeviceIdType.MESH,
                )

        def p1_desc(c):
            return pltpu.make_async_remote_copy(
                src_ref=staged_ref.at[c],
                dst_ref=p1buf_ref.at[my],
                send_sem=ss1.at[c],
                recv_sem=rs1.at[my],
                device_id=(c,),
                device_id_type=pl.DeviceIdType.MESH,
            )

        w1 = w1_ref[...].astype(jnp.bfloat16)
        w2 = w2_ref[...].astype(jnp.bfloat16)
        n_groups = 4
        g_rows = M // n_groups
        g_chunks = N_DEV // n_groups
        zg = my // g_chunks
        low_half = zg <= 1
        for t in range(n_groups):
            g = jnp.where(low_half, 3 - t, t)
            xb = x_ref[pl.ds(g * g_rows, g_rows), :].astype(jnp.bfloat16)
            h = jnp.dot(xb, w1, preferred_element_type=jnp.float32)
            hb = jnp.maximum(h, 0.0).astype(jnp.bfloat16)
            part = jnp.dot(hb, w2, preferred_element_type=jnp.float32)
            staged_ref[pl.ds(g * g_chunks, g_chunks)] = (
                part.astype(jnp.bfloat16).reshape(g_chunks, ROWS, N)
            )
            for j in range(g_chunks):
                c = g * g_chunks + j
                @pl.when(c != my)
                def _():
                    pl.semaphore_wait(entry_sems.at[c], 1)
                    p1_desc(c).start()
        p1buf_ref[pl.ds(my, 1)] = staged_ref[pl.ds(my, 1)]

        def p1_recv_desc(s):
            return pltpu.make_async_remote_copy(
                src_ref=staged_ref.at[s],
                dst_ref=p1buf_ref.at[s],
                send_sem=ss1.at[s],
                recv_sem=rs1.at[s],
                device_id=(s,),
                device_id_type=pl.DeviceIdType.MESH,
            )

        for s in range(N_DEV):
            @pl.when(s != my)
            def _():
                p1_recv_desc(s).wait_recv()

        chunk = jnp.sum(p1buf_ref[...].astype(jnp.float32), axis=0)
        ag_staged_ref[...] = chunk.astype(jnp.bfloat16)

        def p2_desc(c):
            return pltpu.make_async_remote_copy(
                src_ref=ag_staged_ref,
                dst_ref=ag_buf_ref.at[my],
                send_sem=ss2.at[c],
                recv_sem=rs2.at[my],
                device_id=(c,),
                device_id_type=pl.DeviceIdType.MESH,
            )

        for c in range(N_DEV):
            @pl.when(c != my)
            def _():
                p2_desc(c).start()
        ag_buf_ref[pl.ds(my, 1)] = ag_staged_ref[...].reshape(1, ROWS, N)

        def p2_recv_desc(s):
            return pltpu.make_async_remote_copy(
                src_ref=ag_staged_ref,
                dst_ref=ag_buf_ref.at[s],
                send_sem=ss2.at[s],
                recv_sem=rs2.at[s],
                device_id=(s,),
                device_id_type=pl.DeviceIdType.MESH,
            )

        for s in range(N_DEV):
            @pl.when(s != my)
            def _():
                p2_recv_desc(s).wait_recv()
            out_ref[pl.ds(s * ROWS, ROWS), :] = ag_buf_ref[s].astype(jnp.float32)

        for s in range(N_DEV):
            @pl.when(s != my)
            def _():
                p1_recv_desc(s).wait_send()
                p2_recv_desc(s).wait_send()

    return pl.pallas_call(
        body,
        out_shape=jax.ShapeDtypeStruct((M, N), jnp.float32),
        in_specs=[
            pl.BlockSpec(memory_space=pltpu.VMEM),
            pl.BlockSpec(memory_space=pltpu.VMEM),
            pl.BlockSpec(memory_space=pltpu.VMEM),
        ],
        out_specs=pl.BlockSpec(memory_space=pltpu.VMEM),
        scratch_shapes=[
            pltpu.VMEM((N_DEV, ROWS, N), jnp.bfloat16),
            pltpu.VMEM((N_DEV, ROWS, N), jnp.bfloat16),
            pltpu.VMEM((ROWS, N), jnp.bfloat16),
            pltpu.VMEM((N_DEV, ROWS, N), jnp.bfloat16),
            pltpu.SemaphoreType.DMA((N_DEV,)),
            pltpu.SemaphoreType.DMA((N_DEV,)),
            pltpu.SemaphoreType.DMA((N_DEV,)),
            pltpu.SemaphoreType.DMA((N_DEV,)),
            pltpu.SemaphoreType.REGULAR((N_DEV,)),
        ],
        compiler_params=pltpu.CompilerParams(collective_id=0),
    )(x, W1, W2)


# device time: 23342 ns/iter; 1.0956x vs baseline; 1.0599x over previous
import jax
import jax.numpy as jnp
from jax import lax
from jax.experimental import pallas as pl
from jax.experimental.pallas import tpu as pltpu

N_DEV = 16
M = 512
N = 512
ROWS = M // N_DEV


def kernel(x, W1, W2):
    def body(
        x_ref,
        w1_ref,
        w2_ref,
        out_ref,
        staged_ref,
        p1buf_ref,
        foldbuf_ref,
        ag_staged_ref,
        ag_buf_ref,
        ss1, rs1, ss2, rs2,
        ssF, rsF,
        entry_sems,
    ):
        my = lax.axis_index("i")

        barrier_sem = pltpu.get_barrier_semaphore()
        pl.semaphore_signal(barrier_sem, inc=1)
        pl.semaphore_wait(barrier_sem, 1)

        for c in range(N_DEV):
            @pl.when(c != my)
            def _():
                pl.semaphore_signal(
                    entry_sems.at[my],
                    inc=1,
                    device_id=(c,),
                    device_id_type=pl.DeviceIdType.MESH,
                )

        partner = my ^ 4
        nb = (my // 8) * 8
        fq = jnp.where(my < 8, 8, 0)
        zbit = (my // 4) % 2
        kb = fq + zbit * 4
        gb = fq + (1 - zbit) * 4

        def p1_desc(c):
            return pltpu.make_async_remote_copy(
                src_ref=staged_ref.at[c],
                dst_ref=p1buf_ref.at[my],
                send_sem=ss1.at[c],
                recv_sem=rs1.at[my],
                device_id=(c,),
                device_id_type=pl.DeviceIdType.MESH,
            )

        fold_desc = pltpu.make_async_remote_copy(
            src_ref=staged_ref.at[pl.ds(gb, 4)],
            dst_ref=foldbuf_ref,
            send_sem=ssF,
            recv_sem=rsF,
            device_id=(partner,),
            device_id_type=pl.DeviceIdType.MESH,
        )

        w1 = w1_ref[...].astype(jnp.bfloat16)
        w2 = w2_ref[...].astype(jnp.bfloat16)
        n_groups = 4
        g_rows = M // n_groups
        g_chunks = N_DEV // n_groups
        low_half = my < 8
        for t in range(n_groups):
            g = jnp.where(low_half, 3 - t, t)
            xb = x_ref[pl.ds(g * g_rows, g_rows), :].astype(jnp.bfloat16)
            h = jnp.dot(xb, w1, preferred_element_type=jnp.float32)
            hb = jnp.maximum(h, 0.0).astype(jnp.bfloat16)
            part = jnp.dot(hb, w2, preferred_element_type=jnp.float32)
            staged_ref[pl.ds(g * g_chunks, g_chunks)] = (
                part.astype(jnp.bfloat16).reshape(g_chunks, ROWS, N)
            )
            if t == 1:
                pl.semaphore_wait(entry_sems.at[partner], 1)
                fold_desc.start()
            if t >= 2:
                for j in range(g_chunks):
                    c = g * g_chunks + j
                    @pl.when(jnp.logical_and(c != my, c != partner))
                    def _():
                        pl.semaphore_wait(entry_sems.at[c], 1)
                    @pl.when(c != my)
                    def _():
                        p1_desc(c).start()

        p1buf_ref[pl.ds(my, 1)] = staged_ref[pl.ds(my, 1)]
        p1buf_ref[pl.ds(gb, 4)] = jnp.zeros((4, ROWS, N), jnp.bfloat16)

        fold_desc.wait_recv()
        staged_ref[pl.ds(kb, 4)] = (
            staged_ref[pl.ds(kb, 4)].astype(jnp.float32)
            + foldbuf_ref[...].astype(jnp.float32)
        ).astype(jnp.bfloat16)
        for j in range(g_chunks):
            c = kb + j
            pl.semaphore_wait(entry_sems.at[c], 1)
            p1_desc(c).start()

        def p1_recv_desc(s):
            return pltpu.make_async_remote_copy(
                src_ref=staged_ref.at[s],
                dst_ref=p1buf_ref.at[s],
                send_sem=ss1.at[s],
                recv_sem=rs1.at[s],
                device_id=(s,),
                device_id_type=pl.DeviceIdType.MESH,
            )

        def p1_active(s):
            near = (s // 8) * 8 == nb
            far = (s // 4) * 4 == kb
            return jnp.logical_and(s != my, jnp.logical_or(near, far))

        for s in range(N_DEV):
            @pl.when(p1_active(s))
            def _():
                p1_recv_desc(s).wait_recv()

        chunk = jnp.sum(p1buf_ref[...].astype(jnp.float32), axis=0)
        ag_staged_ref[...] = chunk.astype(jnp.bfloat16)

        def p2_desc(c):
            return pltpu.make_async_remote_copy(
                src_ref=ag_staged_ref,
                dst_ref=ag_buf_ref.at[my],
                send_sem=ss2.at[c],
                recv_sem=rs2.at[my],
                device_id=(c,),
                device_id_type=pl.DeviceIdType.MESH,
            )

        for c in range(N_DEV):
            @pl.when((c // 4) * 4 == gb)
            def _():
                pl.semaphore_wait(entry_sems.at[c], 1)
            @pl.when(c != my)
            def _():
                p2_desc(c).start()
        ag_buf_ref[pl.ds(my, 1)] = ag_staged_ref[...].reshape(1, ROWS, N)

        def p2_recv_desc(s):
            return pltpu.make_async_remote_copy(
                src_ref=ag_staged_ref,
                dst_ref=ag_buf_ref.at[s],
                send_sem=ss2.at[s],
                recv_sem=rs2.at[s],
                device_id=(s,),
                device_id_type=pl.DeviceIdType.MESH,
            )

        for s in range(N_DEV):
            @pl.when(s != my)
            def _():
                p2_recv_desc(s).wait_recv()
            out_ref[pl.ds(s * ROWS, ROWS), :] = ag_buf_ref[s].astype(jnp.float32)

        fold_desc.wait_send()
        for s in range(N_DEV):
            @pl.when(p1_active(s))
            def _():
                p1_recv_desc(s).wait_send()
            @pl.when(s != my)
            def _():
                p2_recv_desc(s).wait_send()

    return pl.pallas_call(
        body,
        out_shape=jax.ShapeDtypeStruct((M, N), jnp.float32),
        in_specs=[
            pl.BlockSpec(memory_space=pltpu.VMEM),
            pl.BlockSpec(memory_space=pltpu.VMEM),
            pl.BlockSpec(memory_space=pltpu.VMEM),
        ],
        out_specs=pl.BlockSpec(memory_space=pltpu.VMEM),
        scratch_shapes=[
            pltpu.VMEM((N_DEV, ROWS, N), jnp.bfloat16),
            pltpu.VMEM((N_DEV, ROWS, N), jnp.bfloat16),
            pltpu.VMEM((4, ROWS, N), jnp.bfloat16),
            pltpu.VMEM((ROWS, N), jnp.bfloat16),
            pltpu.VMEM((N_DEV, ROWS, N), jnp.bfloat16),
            pltpu.SemaphoreType.DMA((N_DEV,)),
            pltpu.SemaphoreType.DMA((N_DEV,)),
            pltpu.SemaphoreType.DMA((N_DEV,)),
            pltpu.SemaphoreType.DMA((N_DEV,)),
            pltpu.SemaphoreType.DMA,
            pltpu.SemaphoreType.DMA,
            pltpu.SemaphoreType.REGULAR((N_DEV,)),
        ],
        compiler_params=pltpu.CompilerParams(collective_id=0),
    )(x, W1, W2)
